# Initial kernel scaffold; baseline (speedup 1.0000x reference)
import jax
import jax.numpy as jnp
from jax.experimental import pallas as pl


def kernel(v, pos, edges, offsets_real, lin_w, mlp_w1, mlp_b1, mlp_w2, mlp_b2, v_w1, v_b1, v_w2, v_b2):
    raise NotImplementedError("write your pallas kernel here")


# speedup vs baseline: 2.5673x; 2.5673x over previous
import math
import jax
import jax.numpy as jnp
from jax import lax
from jax.experimental import pallas as pl
from jax.experimental.pallas import tpu as pltpu
from jax.experimental.pallas import tpu_sc as plsc

CUTOFF = 5.0
LN2 = math.log(2.0)
NC = 2
NS = 16
NW = NC * NS
CHUNK = 128
LANE = 16


def _softplus(x):
    return jnp.maximum(x, 0.0) + jnp.log(1.0 + jnp.exp(-jnp.abs(x)))


def _largest_div(n, cap, mult=1):
    for d in range(min(n, cap), 0, -1):
        if n % d == 0 and d % mult == 0:
            return d
    return 1


def _d2_call(row3, col3, ox3, oy3, oz3, px, py, pz):
    NWv, GRP, _ = row3.shape
    mesh = plsc.VectorSubcoreMesh(core_axis_name="c", subcore_axis_name="s")

    def body(row_hbm, col_hbm, ox_hbm, oy_hbm, oz_hbm, px_hbm, py_hbm, pz_hbm,
             d2_hbm, px_v, py_v, pz_v, row_v, col_v, ox_v, oy_v, oz_v, d2_v):
        c = lax.axis_index("c")
        s = lax.axis_index("s")
        wid = s * NC + c
        pltpu.sync_copy(px_hbm, px_v)
        pltpu.sync_copy(py_hbm, py_v)
        pltpu.sync_copy(pz_hbm, pz_v)
        pltpu.sync_copy(row_hbm.at[wid], row_v)
        pltpu.sync_copy(col_hbm.at[wid], col_v)
        pltpu.sync_copy(ox_hbm.at[wid], ox_v)
        pltpu.sync_copy(oy_hbm.at[wid], oy_v)
        pltpu.sync_copy(oz_hbm.at[wid], oz_v)

        def step(i, _):
            r = row_v[i]
            cc = col_v[i]
            rx = plsc.load_gather(px_v, [r])
            ry = plsc.load_gather(py_v, [r])
            rz = plsc.load_gather(pz_v, [r])
            cx = plsc.load_gather(px_v, [cc])
            cy = plsc.load_gather(py_v, [cc])
            cz = plsc.load_gather(pz_v, [cc])
            dx = cx + ox_v[i] - rx
            dy = cy + oy_v[i] - ry
            dz = cz + oz_v[i] - rz
            d2_v[i] = dx * dx + dy * dy + dz * dz
            return 0

        lax.fori_loop(0, GRP, step, 0)
        pltpu.sync_copy(d2_v, d2_hbm.at[wid])

    Np = px.shape[0]
    k = pl.kernel(
        body,
        out_type=jax.ShapeDtypeStruct((NWv, GRP, LANE), jnp.float32),
        mesh=mesh,
        compiler_params=pltpu.CompilerParams(needs_layout_passes=False,
                                             use_tc_tiling_on_sc=False),
        scratch_types=[
            pltpu.VMEM((Np,), jnp.float32),
            pltpu.VMEM((Np,), jnp.float32),
            pltpu.VMEM((Np,), jnp.float32),
            pltpu.VMEM((GRP, LANE), jnp.int32),
            pltpu.VMEM((GRP, LANE), jnp.int32),
            pltpu.VMEM((GRP, LANE), jnp.float32),
            pltpu.VMEM((GRP, LANE), jnp.float32),
            pltpu.VMEM((GRP, LANE), jnp.float32),
            pltpu.VMEM((GRP, LANE), jnp.float32),
        ],
    )
    return k(row3, col3, ox3, oy3, oz3, px, py, pz)


def _filters_call(d2r, w1t, b1r, w2t, b2r, E, L, G, GP, H, E_PAD):
    BE = 4096
    nblk = E_PAD // BE

    def body(d2_ref, w1t_ref, b1_ref, w2t_ref, b2_ref, wout_ref):
        b = pl.program_id(1)
        d2 = d2_ref[...].reshape(BE)
        dist = jnp.sqrt(d2)
        step = CUTOFF / (G - 1)
        offs = lax.broadcasted_iota(jnp.int32, (BE, GP), 1).astype(jnp.float32) * step
        coeff = -0.5 / (step * step)
        emb = jnp.exp(coeff * (dist[:, None] - offs) ** 2)
        h1 = jnp.dot(emb, w1t_ref[0], preferred_element_type=jnp.float32)
        h1 = _softplus(h1 + b1_ref[0]) - LN2
        Wf = jnp.dot(h1, w2t_ref[0], preferred_element_type=jnp.float32) + b2_ref[0]
        Cc = 0.5 * (jnp.cos(dist * (math.pi / CUTOFF)) + 1.0)
        eidx = b * BE + lax.broadcasted_iota(jnp.int32, (BE,), 0)
        Cc = jnp.where(eidx < E, Cc, 0.0)
        wout_ref[0] = Wf * Cc[:, None]

    grid = (L, nblk)
    return pl.pallas_call(
        body,
        grid=grid,
        in_specs=[
            pl.BlockSpec((BE // 512, 512), lambda l, b: (b, 0)),
            pl.BlockSpec((1, GP, H), lambda l, b: (l, 0, 0)),
            pl.BlockSpec((1, 1, H), lambda l, b: (l, 0, 0)),
            pl.BlockSpec((1, H, H), lambda l, b: (l, 0, 0)),
            pl.BlockSpec((1, 1, H), lambda l, b: (l, 0, 0)),
        ],
        out_specs=pl.BlockSpec((1, BE, H), lambda l, b: (l, b, 0)),
        out_shape=jax.ShapeDtypeStruct((L, E_PAD, H), jnp.float32),
    )(d2r, w1t, b1r, w2t, b2r)


def _message_call(w_e, vh, rowC, colC, N, H, CPT, TPW):
    mesh = plsc.VectorSubcoreMesh(core_axis_name="c", subcore_axis_name="s")
    ROWS_PT = N // NS
    ZC = _largest_div(ROWS_PT, CHUNK)

    def body(w_hbm, vh_hbm, row_hbm, col_hbm, out_hbm,
             accum_sh, rowi_v, coli_v, w_v, vhg_v, sem_w, sem_g, sem_r, sem_c):
        c = lax.axis_index("c")
        s = lax.axis_index("s")
        wid = s * NC + c
        zz = jnp.zeros((LANE,), jnp.float32)

        def zrow(i, _):
            for j in range(H // LANE):
                w_v[i, pl.ds(j * LANE, LANE)] = zz
            return 0

        lax.fori_loop(0, CHUNK, zrow, 0)
        rbase = s * ROWS_PT
        for t in range(ROWS_PT // ZC):
            pltpu.sync_copy(w_v.at[pl.ds(0, ZC)],
                            accum_sh.at[pl.ds(rbase + t * ZC, ZC)])
        plsc.subcore_barrier()
        ebase = wid * TPW

        def chunk(k, _):
            dr = pltpu.async_copy(row_hbm.at[wid].at[pl.ds(k, 1)], rowi_v, sem_r)
            dc = pltpu.async_copy(col_hbm.at[wid].at[pl.ds(k, 1)], coli_v, sem_c)
            dw = pltpu.async_copy(w_hbm.at[pl.ds(ebase + k * CHUNK, CHUNK)], w_v, sem_w)
            dr.wait()
            dg = pltpu.async_copy(vh_hbm.at[rowi_v.at[0]], vhg_v, sem_g)
            dw.wait()
            dg.wait()

            def mrow(i, _):
                for j in range(H // LANE):
                    sl = pl.ds(j * LANE, LANE)
                    w_v[i, sl] = w_v[i, sl] * vhg_v[i, sl]
                return 0

            lax.fori_loop(0, CHUNK, mrow, 0)
            dc.wait()
            pltpu.sync_copy(w_v, accum_sh.at[coli_v.at[0]], add=True)
            return 0

        lax.fori_loop(0, CPT, chunk, 0)
        plsc.subcore_barrier()
        pltpu.sync_copy(accum_sh.at[pl.ds(rbase, ROWS_PT)],
                        out_hbm.at[c, pl.ds(rbase, ROWS_PT)])

    k = pl.kernel(
        body,
        out_type=jax.ShapeDtypeStruct((NC, N, H), jnp.float32),
        mesh=mesh,
        compiler_params=pltpu.CompilerParams(needs_layout_passes=False,
                                             use_tc_tiling_on_sc=False),
        scratch_types=[
            pltpu.VMEM_SHARED((N, H), jnp.float32),
            pltpu.VMEM((1, CHUNK), jnp.int32),
            pltpu.VMEM((1, CHUNK), jnp.int32),
            pltpu.VMEM((CHUNK, H), jnp.float32),
            pltpu.VMEM((CHUNK, H), jnp.float32),
            pltpu.SemaphoreType.DMA,
            pltpu.SemaphoreType.DMA,
            pltpu.SemaphoreType.DMA,
            pltpu.SemaphoreType.DMA,
        ],
    )
    return k(w_e, vh, rowC, colC)


def _update_call(part, v, w1t, b1, w2t, b2, lint, N, H):
    BN = _largest_div(N, 1024, mult=8)

    def body(p_ref, v_ref, w1_ref, b1_ref, w2_ref, b2_ref, lt_ref, vn_ref, vh_ref):
        out = p_ref[0] + p_ref[1]
        h = _softplus(jnp.dot(out, w1_ref[...], preferred_element_type=jnp.float32)
                      + b1_ref[...]) - LN2
        upd = jnp.dot(h, w2_ref[...], preferred_element_type=jnp.float32) + b2_ref[...]
        vn = v_ref[...] + upd
        vn_ref[...] = vn
        vh_ref[...] = jnp.dot(vn, lt_ref[...], preferred_element_type=jnp.float32)

    grid = (N // BN,)
    return pl.pallas_call(
        body,
        grid=grid,
        in_specs=[
            pl.BlockSpec((2, BN, H), lambda b: (0, b, 0)),
            pl.BlockSpec((BN, H), lambda b: (b, 0)),
            pl.BlockSpec((H, H), lambda b: (0, 0)),
            pl.BlockSpec((1, H), lambda b: (0, 0)),
            pl.BlockSpec((H, H), lambda b: (0, 0)),
            pl.BlockSpec((1, H), lambda b: (0, 0)),
            pl.BlockSpec((H, H), lambda b: (0, 0)),
        ],
        out_specs=[
            pl.BlockSpec((BN, H), lambda b: (b, 0)),
            pl.BlockSpec((BN, H), lambda b: (b, 0)),
        ],
        out_shape=[
            jax.ShapeDtypeStruct((N, H), jnp.float32),
            jax.ShapeDtypeStruct((N, H), jnp.float32),
        ],
    )(part, v, w1t, b1, w2t, b2, lint)


def _vh0_call(v, lint, N, H):
    BN = _largest_div(N, 1024, mult=8)

    def body(v_ref, lt_ref, vh_ref):
        vh_ref[...] = jnp.dot(v_ref[...], lt_ref[...],
                              preferred_element_type=jnp.float32)

    return pl.pallas_call(
        body,
        grid=(N // BN,),
        in_specs=[
            pl.BlockSpec((BN, H), lambda b: (b, 0)),
            pl.BlockSpec((H, H), lambda b: (0, 0)),
        ],
        out_specs=pl.BlockSpec((BN, H), lambda b: (b, 0)),
        out_shape=jax.ShapeDtypeStruct((N, H), jnp.float32),
    )(v, lint)


def kernel(v, pos, edges, offsets_real, lin_w, mlp_w1, mlp_b1, mlp_w2, mlp_b2,
           v_w1, v_b1, v_w2, v_b2):
    N, H = v.shape
    L, FLT, G = mlp_w1.shape
    E = edges.shape[1]
    assert N % NS == 0 and H % LANE == 0

    CPT = -(-E // (NW * CHUNK))
    TPW = CPT * CHUNK
    E_PAD = NW * TPW
    GRP = TPW // LANE
    GP = -(-G // 8) * 8

    f32 = jnp.float32
    row = jnp.pad(edges[0], (0, E_PAD - E)).astype(jnp.int32)
    col = jnp.pad(edges[1], (0, E_PAD - E)).astype(jnp.int32)
    offp = jnp.pad(offsets_real, ((0, E_PAD - E), (0, 0))).astype(f32)
    row3 = row.reshape(NW, GRP, LANE)
    col3 = col.reshape(NW, GRP, LANE)
    ox3 = offp[:, 0].reshape(NW, GRP, LANE)
    oy3 = offp[:, 1].reshape(NW, GRP, LANE)
    oz3 = offp[:, 2].reshape(NW, GRP, LANE)
    px = pos[:, 0].astype(f32)
    py = pos[:, 1].astype(f32)
    pz = pos[:, 2].astype(f32)

    d2 = _d2_call(row3, col3, ox3, oy3, oz3, px, py, pz)
    d2r = d2.reshape(E_PAD // 512, 512)

    w1t = jnp.pad(jnp.swapaxes(mlp_w1, 1, 2), ((0, 0), (0, GP - G), (0, 0))).astype(f32)
    b1r = mlp_b1.reshape(L, 1, FLT).astype(f32)
    w2t = jnp.swapaxes(mlp_w2, 1, 2).astype(f32)
    b2r = mlp_b2.reshape(L, 1, FLT).astype(f32)
    W_all = _filters_call(d2r, w1t, b1r, w2t, b2r, E, L, G, GP, H, E_PAD)

    rowC = row.reshape(NW, CPT, CHUNK)
    colC = col.reshape(NW, CPT, CHUNK)

    v = v.astype(f32)
    vh = _vh0_call(v, jnp.swapaxes(lin_w[0], 0, 1).astype(f32), N, H)
    for l in range(L):
        part = _message_call(W_all[l], vh, rowC, colC, N, H, CPT, TPW)
        lint = jnp.swapaxes(lin_w[(l + 1) % L], 0, 1).astype(f32)
        v, vh = _update_call(part, v,
                             jnp.swapaxes(v_w1[l], 0, 1).astype(f32),
                             v_b1[l].reshape(1, H).astype(f32),
                             jnp.swapaxes(v_w2[l], 0, 1).astype(f32),
                             v_b2[l].reshape(1, H).astype(f32),
                             lint, N, H)
    return v
